# baseline (speedup 1.0000x reference)
import jax
import jax.numpy as jnp
import numpy as np
from jax.experimental import pallas as pl

_BN = 1.0 / np.sqrt(1.0 + 1e-5)


def _sa_mlp_max_body(x_ref, *refs):
    o_ref = refs[-1]
    wbs = refs[:-1]
    acc = None
    for k in range(x_ref.shape[0]):
        h = x_ref[k]
        for j in range(0, len(wbs), 2):
            w = wbs[j][...]
            b = wbs[j + 1][...]
            h = jnp.maximum((jnp.dot(h, w, preferred_element_type=jnp.float32) + b) * _BN, 0.0)
        acc = h if acc is None else jnp.maximum(acc, h)
    o_ref[...] = acc


def _sa_mlp_max(x, ws, bs):
    K, S, Cin = x.shape
    args = [x]
    for w, b in zip(ws, bs):
        args.append(w.T)
        args.append(b[None, :])
    Cout = ws[-1].shape[0]
    return pl.pallas_call(
        _sa_mlp_max_body,
        out_shape=jax.ShapeDtypeStruct((S, Cout), x.dtype),
    )(*args)


def _mlp_body(x_ref, *refs):
    o_ref = refs[-1]
    wbs = refs[:-1]
    h = x_ref[...]
    for j in range(0, len(wbs), 2):
        w = wbs[j][...]
        b = wbs[j + 1][...]
        h = jnp.maximum((jnp.dot(h, w, preferred_element_type=jnp.float32) + b) * _BN, 0.0)
    o_ref[...] = h


def _mlp(x, ws, bs):
    args = [x]
    for w, b in zip(ws, bs):
        args.append(w.T)
        args.append(b[None, :])
    Cout = ws[-1].shape[0]
    return pl.pallas_call(
        _mlp_body,
        out_shape=jax.ShapeDtypeStruct((x.shape[0], Cout), x.dtype),
    )(*args)


def _head_body(x_ref, w_ref, b_ref, o_ref):
    h = jnp.dot(x_ref[...], w_ref[...], preferred_element_type=jnp.float32) + b_ref[...]
    mean = jnp.mean(h, axis=0, keepdims=True)
    var = jnp.mean((h - mean) ** 2, axis=0, keepdims=True)
    h = (h - mean) * jax.lax.rsqrt(var + 1e-5)
    o_ref[...] = jnp.maximum(h, 0.0)


def _head(x, w, b):
    return pl.pallas_call(
        _head_body,
        out_shape=jax.ShapeDtypeStruct((x.shape[0], w.shape[0]), x.dtype),
    )(x, w.T, b[None, :])


def _square_distance(src, dst):
    return (-2.0 * src @ dst.T
            + jnp.sum(src ** 2, axis=-1)[:, None]
            + jnp.sum(dst ** 2, axis=-1)[None, :])


def _fps(d, npoint):
    N = d.shape[0]

    def body(i, state):
        centroids, distance, farthest = state
        centroids = centroids.at[i].set(farthest)
        distance = jnp.minimum(distance, d[farthest])
        farthest = jnp.argmax(distance).astype(jnp.int32)
        return centroids, distance, farthest

    centroids = jnp.zeros((npoint,), dtype=jnp.int32)
    distance = jnp.full((N,), 1e10, dtype=d.dtype)
    farthest = jnp.array(0, dtype=jnp.int32)
    centroids, _, _ = jax.lax.fori_loop(0, npoint, body, (centroids, distance, farthest))
    return centroids


def _query_ball(radius, nsample, sqrdists):
    S, N = sqrdists.shape
    group_idx = jnp.broadcast_to(jnp.arange(N, dtype=jnp.int32), (S, N))
    group_idx = jnp.where(sqrdists > radius ** 2, N, group_idx)
    group_idx = jnp.sort(group_idx, axis=-1)[:, :nsample]
    group_first = jnp.broadcast_to(group_idx[:, :1], (S, nsample))
    group_idx = jnp.where(group_idx == N, group_first, group_idx)
    return group_idx


def _set_abstraction(xyz_t, pts_t, dists, radius, nsample, ws, bs):
    N = xyz_t.shape[0]
    npoint = N // 4
    fps_idx = _fps(dists, npoint)
    new_xyz = xyz_t[fps_idx]
    sqr = dists[fps_idx]
    group_idx = _query_ball(radius, nsample, sqr)
    grouped = jnp.concatenate(
        [pts_t[group_idx], xyz_t[group_idx] - new_xyz[:, None, :]], axis=-1)
    x = jnp.transpose(grouped, (1, 0, 2))
    new_points = _sa_mlp_max(x, ws, bs)
    return new_xyz, new_points, fps_idx


def _feature_prop(pts1, pts2, fps_idx, dists, ws, bs):
    cross = dists[:, fps_idx]
    neg_d, idx = jax.lax.top_k(-cross, 3)
    d = -neg_d
    recip = 1.0 / (d + 1e-8)
    weight = recip / jnp.sum(recip, axis=-1, keepdims=True)
    interp = jnp.sum(pts2[idx] * weight[..., None], axis=1)
    x = interp if pts1 is None else jnp.concatenate([pts1, interp], axis=-1)
    return _mlp(x, ws, bs)


def kernel(vertices, normals,
           sa1_w0, sa1_b0, sa1_w1, sa1_b1, sa1_w2, sa1_b2,
           sa2_w0, sa2_b0, sa2_w1, sa2_b1, sa2_w2, sa2_b2,
           sa3_w0, sa3_b0, sa3_w1, sa3_b1, sa3_w2, sa3_b2,
           sa4_w0, sa4_b0, sa4_w1, sa4_b1, sa4_w2, sa4_b2,
           fp4_w0, fp4_b0, fp4_w1, fp4_b1,
           fp3_w0, fp3_b0, fp3_w1, fp3_b1,
           fp2_w0, fp2_b0, fp2_w1, fp2_b1,
           fp1_w0, fp1_b0, fp1_w1, fp1_b1, fp1_w2, fp1_b2,
           conv1_w0, conv1_b0):
    C, N, _ = vertices.shape
    total = C * N
    dists = jnp.ones((total, total), dtype=vertices.dtype)
    for i in range(C):
        v = vertices[i]
        dists = dists.at[i * N:(i + 1) * N, i * N:(i + 1) * N].set(
            _square_distance(v, v))
    verts_flat = vertices.reshape(total, 3)
    pts0 = jnp.concatenate([verts_flat, normals], axis=-1)
    xyz0 = verts_flat

    l1_xyz, l1_points, fi1 = _set_abstraction(
        xyz0, pts0, dists, 0.06, 32, [sa1_w0, sa1_w1, sa1_w2], [sa1_b0, sa1_b1, sa1_b2])
    dists2 = dists[fi1][:, fi1]
    l2_xyz, l2_points, fi2 = _set_abstraction(
        l1_xyz, l1_points, dists2, 0.1, 32, [sa2_w0, sa2_w1, sa2_w2], [sa2_b0, sa2_b1, sa2_b2])
    dists3 = dists2[fi2][:, fi2]
    l3_xyz, l3_points, fi3 = _set_abstraction(
        l2_xyz, l2_points, dists3, 0.14, 32, [sa3_w0, sa3_w1, sa3_w2], [sa3_b0, sa3_b1, sa3_b2])
    dists4 = dists3[fi3][:, fi3]
    l4_xyz, l4_points, fi4 = _set_abstraction(
        l3_xyz, l3_points, dists4, 0.18, 32, [sa4_w0, sa4_w1, sa4_w2], [sa4_b0, sa4_b1, sa4_b2])

    l3_points = _feature_prop(l3_points, l4_points, fi4, dists4,
                              [fp4_w0, fp4_w1], [fp4_b0, fp4_b1])
    l2_points = _feature_prop(l2_points, l3_points, fi3, dists3,
                              [fp3_w0, fp3_w1], [fp3_b0, fp3_b1])
    l1_points = _feature_prop(l1_points, l2_points, fi2, dists2,
                              [fp2_w0, fp2_w1], [fp2_b0, fp2_b1])
    l0_points = _feature_prop(None, l1_points, fi1, dists,
                              [fp1_w0, fp1_w1, fp1_w2], [fp1_b0, fp1_b1, fp1_b2])

    feats = _head(l0_points, conv1_w0, conv1_b0)
    return feats[None]


# speedup vs baseline: 3.4744x; 3.4744x over previous
import jax
import jax.numpy as jnp
import numpy as np
from jax.experimental import pallas as pl
from jax.experimental.pallas import tpu as pltpu

_BN = 1.0 / np.sqrt(1.0 + 1e-5)


def _fps1_body(dloc_ref, out_ref, dist_ref):
    B, Nl = dist_ref.shape
    dist_ref[...] = jnp.full((B, Nl), 1e10, jnp.float32)
    lin = (jax.lax.broadcasted_iota(jnp.int32, (B, Nl), 0) * Nl
           + jax.lax.broadcasted_iota(jnp.int32, (B, Nl), 1))
    rowid = jax.lax.broadcasted_iota(jnp.int32, (B, Nl), 0)

    def body(i, f):
        out_ref[pl.ds(i, 1), :] = jnp.full((1, 1), f, jnp.int32)
        b = f // Nl
        row = dloc_ref[pl.ds(f, 1), :]
        full_row = jnp.where(rowid == b, jnp.broadcast_to(row, (B, Nl)),
                             jnp.float32(1.0))
        newd = jnp.minimum(dist_ref[...], full_row)
        dist_ref[...] = newd
        m = jnp.max(newd)
        return jnp.min(jnp.where(newd == m, lin, jnp.int32(2 ** 30))).astype(jnp.int32)

    jax.lax.fori_loop(0, out_ref.shape[0], body, jnp.int32(0))


def _fps1(dloc, nblocks, npoint):
    total, nl = dloc.shape
    out = pl.pallas_call(
        _fps1_body,
        out_shape=jax.ShapeDtypeStruct((npoint, 1), jnp.int32),
        scratch_shapes=[pltpu.VMEM((nblocks, nl), jnp.float32)],
    )(dloc)
    return out.reshape(-1)


def _fps_small_body(d_ref, out_ref, dist_ref):
    N = d_ref.shape[0]
    dist_ref[...] = jnp.full((1, N), 1e10, jnp.float32)
    lin = jax.lax.broadcasted_iota(jnp.int32, (1, N), 1)

    def body(i, f):
        out_ref[pl.ds(i, 1), :] = jnp.full((1, 1), f, jnp.int32)
        row = d_ref[pl.ds(f, 1), :]
        newd = jnp.minimum(dist_ref[...], row)
        dist_ref[...] = newd
        m = jnp.max(newd)
        return jnp.min(jnp.where(newd == m, lin, jnp.int32(2 ** 30))).astype(jnp.int32)

    jax.lax.fori_loop(0, out_ref.shape[0], body, jnp.int32(0))


def _fps_small(d, npoint):
    N = d.shape[0]
    out = pl.pallas_call(
        _fps_small_body,
        out_shape=jax.ShapeDtypeStruct((npoint, 1), jnp.int32),
        scratch_shapes=[pltpu.VMEM((1, N), jnp.float32)],
    )(d)
    return out.reshape(-1)


def _sa_mlp_max_body(x_ref, *refs):
    o_ref = refs[-1]
    wbs = refs[:-1]
    acc = None
    for k in range(x_ref.shape[0]):
        h = x_ref[k]
        for j in range(0, len(wbs), 2):
            w = wbs[j][...]
            b = wbs[j + 1][...]
            h = jnp.maximum((jnp.dot(h, w, preferred_element_type=jnp.float32) + b) * _BN, 0.0)
        acc = h if acc is None else jnp.maximum(acc, h)
    o_ref[...] = acc


def _sa_mlp_max(x, ws, bs):
    K, S, Cin = x.shape
    args = [x]
    for w, b in zip(ws, bs):
        args.append(w.T)
        args.append(b[None, :])
    Cout = ws[-1].shape[0]
    return pl.pallas_call(
        _sa_mlp_max_body,
        out_shape=jax.ShapeDtypeStruct((S, Cout), x.dtype),
    )(*args)


def _mlp_body(x_ref, *refs):
    o_ref = refs[-1]
    wbs = refs[:-1]
    h = x_ref[...]
    for j in range(0, len(wbs), 2):
        w = wbs[j][...]
        b = wbs[j + 1][...]
        h = jnp.maximum((jnp.dot(h, w, preferred_element_type=jnp.float32) + b) * _BN, 0.0)
    o_ref[...] = h


def _mlp(x, ws, bs):
    args = [x]
    for w, b in zip(ws, bs):
        args.append(w.T)
        args.append(b[None, :])
    Cout = ws[-1].shape[0]
    return pl.pallas_call(
        _mlp_body,
        out_shape=jax.ShapeDtypeStruct((x.shape[0], Cout), x.dtype),
    )(*args)


def _head_body(x_ref, w_ref, b_ref, o_ref):
    h = jnp.dot(x_ref[...], w_ref[...], preferred_element_type=jnp.float32) + b_ref[...]
    mean = jnp.mean(h, axis=0, keepdims=True)
    var = jnp.mean((h - mean) ** 2, axis=0, keepdims=True)
    h = (h - mean) * jax.lax.rsqrt(var + 1e-5)
    o_ref[...] = jnp.maximum(h, 0.0)


def _head(x, w, b):
    return pl.pallas_call(
        _head_body,
        out_shape=jax.ShapeDtypeStruct((x.shape[0], w.shape[0]), x.dtype),
    )(x, w.T, b[None, :])


def _square_distance(src, dst):
    return (-2.0 * src @ dst.T
            + jnp.sum(src ** 2, axis=-1)[:, None]
            + jnp.sum(dst ** 2, axis=-1)[None, :])


def _query_ball(radius, nsample, sqrdists):
    S, N = sqrdists.shape
    group_idx = jnp.broadcast_to(jnp.arange(N, dtype=jnp.int32), (S, N))
    group_idx = jnp.where(sqrdists > radius ** 2, N, group_idx)
    group_idx = jnp.sort(group_idx, axis=-1)[:, :nsample]
    group_first = jnp.broadcast_to(group_idx[:, :1], (S, nsample))
    group_idx = jnp.where(group_idx == N, group_first, group_idx)
    return group_idx


def _set_abstraction(xyz_t, pts_t, dists, radius, nsample, ws, bs):
    N = xyz_t.shape[0]
    npoint = N // 4
    fps_idx = _fps_small(dists, npoint)
    new_xyz = xyz_t[fps_idx]
    sqr = dists[fps_idx]
    group_idx = _query_ball(radius, nsample, sqr)
    grouped = jnp.concatenate(
        [pts_t[group_idx], xyz_t[group_idx] - new_xyz[:, None, :]], axis=-1)
    x = jnp.transpose(grouped, (1, 0, 2))
    new_points = _sa_mlp_max(x, ws, bs)
    return new_xyz, new_points, fps_idx


def _feature_prop(pts1, pts2, cross, ws, bs):
    neg_d, idx = jax.lax.top_k(-cross, 3)
    d = -neg_d
    recip = 1.0 / (d + 1e-8)
    weight = recip / jnp.sum(recip, axis=-1, keepdims=True)
    interp = jnp.sum(pts2[idx] * weight[..., None], axis=1)
    x = interp if pts1 is None else jnp.concatenate([pts1, interp], axis=-1)
    return _mlp(x, ws, bs)


def kernel(vertices, normals,
           sa1_w0, sa1_b0, sa1_w1, sa1_b1, sa1_w2, sa1_b2,
           sa2_w0, sa2_b0, sa2_w1, sa2_b1, sa2_w2, sa2_b2,
           sa3_w0, sa3_b0, sa3_w1, sa3_b1, sa3_w2, sa3_b2,
           sa4_w0, sa4_b0, sa4_w1, sa4_b1, sa4_w2, sa4_b2,
           fp4_w0, fp4_b0, fp4_w1, fp4_b1,
           fp3_w0, fp3_b0, fp3_w1, fp3_b1,
           fp2_w0, fp2_b0, fp2_w1, fp2_b1,
           fp1_w0, fp1_b0, fp1_w1, fp1_b1, fp1_w2, fp1_b2,
           conv1_w0, conv1_b0):
    C, N, _ = vertices.shape
    total = C * N
    dloc = jnp.concatenate(
        [_square_distance(vertices[i], vertices[i]) for i in range(C)], axis=0)
    verts_flat = vertices.reshape(total, 3)
    pts0 = jnp.concatenate([verts_flat, normals], axis=-1)
    xyz0 = verts_flat

    fi1 = _fps1(dloc, C, total // 4)
    new_xyz1 = xyz0[fi1]
    sqr_loc = dloc[fi1]
    gloc = _query_ball(0.06, 32, sqr_loc)
    gidx = jnp.where(gloc == N, total - 1,
                     gloc + ((fi1 // N) * N)[:, None])
    grouped = jnp.concatenate(
        [pts0[gidx], xyz0[gidx] - new_xyz1[:, None, :]], axis=-1)
    l1_points = _sa_mlp_max(jnp.transpose(grouped, (1, 0, 2)),
                            [sa1_w0, sa1_w1, sa1_w2], [sa1_b0, sa1_b1, sa1_b2])
    l1_xyz = new_xyz1

    pb = jnp.arange(total, dtype=jnp.int32) // N
    cb = fi1 // N
    cl = fi1 % N
    cross1 = jnp.where(pb[:, None] == cb[None, :], dloc[:, cl],
                       jnp.float32(1.0))
    dists2 = cross1[fi1]
    l2_xyz, l2_points, fi2 = _set_abstraction(
        l1_xyz, l1_points, dists2, 0.1, 32, [sa2_w0, sa2_w1, sa2_w2], [sa2_b0, sa2_b1, sa2_b2])
    dists3 = dists2[fi2][:, fi2]
    l3_xyz, l3_points, fi3 = _set_abstraction(
        l2_xyz, l2_points, dists3, 0.14, 32, [sa3_w0, sa3_w1, sa3_w2], [sa3_b0, sa3_b1, sa3_b2])
    dists4 = dists3[fi3][:, fi3]
    l4_xyz, l4_points, fi4 = _set_abstraction(
        l3_xyz, l3_points, dists4, 0.18, 32, [sa4_w0, sa4_w1, sa4_w2], [sa4_b0, sa4_b1, sa4_b2])

    l3_points = _feature_prop(l3_points, l4_points, dists4[:, fi4],
                              [fp4_w0, fp4_w1], [fp4_b0, fp4_b1])
    l2_points = _feature_prop(l2_points, l3_points, dists3[:, fi3],
                              [fp3_w0, fp3_w1], [fp3_b0, fp3_b1])
    l1_points = _feature_prop(l1_points, l2_points, dists2[:, fi2],
                              [fp2_w0, fp2_w1], [fp2_b0, fp2_b1])
    l0_points = _feature_prop(None, l1_points, cross1,
                              [fp1_w0, fp1_w1, fp1_w2], [fp1_b0, fp1_b1, fp1_b2])

    feats = _head(l0_points, conv1_w0, conv1_b0)
    return feats[None]
